# baseline (speedup 1.0000x reference)
import functools

import jax
import jax.numpy as jnp
from jax import lax
from jax.experimental import pallas as pl
from jax.experimental.pallas import tpu as pltpu
from jax.experimental.pallas import tpu_sc as plsc

_ROWS_BLOCK = 5000
_GATHER_WINDOW = 128


def _transform_body(t_ref, w_ref, b_ref, o_ref):
    x = t_ref[...]
    row_ids = lax.broadcasted_iota(jnp.int32, x.shape, 0)
    is_row0 = jnp.logical_and(pl.program_id(0) == 0, row_ids == 0)
    x = jnp.where(is_row0, 0.0, x)
    y = lax.dot_general(
        x,
        w_ref[...],
        (((1,), (1,)), ((), ())),
        preferred_element_type=jnp.float32,
        precision=lax.Precision.DEFAULT,
    )
    o_ref[...] = jnp.maximum(y + b_ref[...], 0.0)


def _transform_table(table, W, b):
    num_rows, embed_dim = table.shape
    out_dim = W.shape[0]
    grid = num_rows // _ROWS_BLOCK
    return pl.pallas_call(
        _transform_body,
        grid=(grid,),
        in_specs=[
            pl.BlockSpec((_ROWS_BLOCK, embed_dim), lambda i: (i, 0)),
            pl.BlockSpec((out_dim, embed_dim), lambda i: (0, 0)),
            pl.BlockSpec((1, out_dim), lambda i: (0, 0)),
        ],
        out_specs=pl.BlockSpec((_ROWS_BLOCK, out_dim), lambda i: (i, 0)),
        out_shape=jax.ShapeDtypeStruct((num_rows, out_dim), jnp.float32),
    )(table, W, b.reshape(1, out_dim))


def _sc_gather(t2, idx_flat):
    n = idx_flat.shape[0]
    out_dim = t2.shape[1]
    num_workers = 32
    per_worker = n // num_workers
    chunk = _GATHER_WINDOW
    steps = per_worker // chunk
    idx_blk = 1024
    items_per_blk = idx_blk // chunk
    num_blks = per_worker // idx_blk
    mesh = plsc.VectorSubcoreMesh(core_axis_name="c", subcore_axis_name="s")

    @functools.partial(
        pl.kernel,
        out_type=jax.ShapeDtypeStruct((n, out_dim), jnp.float32),
        mesh=mesh,
        scratch_types=(
            [pltpu.VMEM((idx_blk,), jnp.int32)] * 4
            + [pltpu.VMEM((chunk, out_dim), jnp.float32)] * 4
            + [pltpu.SemaphoreType.DMA] * 12
        ),
    )
    def gather_kernel(t2_hbm, i_hbm, o_hbm,
                      idx0, idx1, idx2, idx3, rows0, rows1, rows2, rows3,
                      isem0, isem1, isem2, isem3,
                      gsem0, gsem1, gsem2, gsem3,
                      ssem0, ssem1, ssem2, ssem3):
        idx_bufs = (idx0, idx1, idx2, idx3)
        rows_bufs = (rows0, rows1, rows2, rows3)
        isems = (isem0, isem1, isem2, isem3)
        gsems = (gsem0, gsem1, gsem2, gsem3)
        ssems = (ssem0, ssem1, ssem2, ssem3)
        nb = 4
        wid = lax.axis_index("s") * 2 + lax.axis_index("c")
        base = wid * per_worker

        def idx_copy(blk, ib):
            return pltpu.make_async_copy(
                i_hbm.at[pl.ds(base + blk * idx_blk, idx_blk)],
                idx_bufs[ib], isems[ib])

        def gather_copy(ib, item_in_blk, rb):
            return pltpu.make_async_copy(
                t2_hbm.at[idx_bufs[ib].at[pl.ds(item_in_blk * chunk, chunk)]],
                rows_bufs[rb], gsems[rb])

        def store_copy(j, rb):
            return pltpu.make_async_copy(
                rows_bufs[rb], o_hbm.at[pl.ds(base + j * chunk, chunk)],
                ssems[rb])

        idx_copy(0, 0).start()

        @pl.loop(0, num_blks, step=4)
        def _(B):
            for ib in range(4):
                blk = B + ib
                idx_copy(blk, ib).wait()

                @pl.when(blk + 1 < num_blks)
                def _():
                    idx_copy(blk + 1, (ib + 1) % 4).start()

                @pl.loop(0, items_per_blk, step=nb)
                def _(q):
                    for rb in range(nb):
                        j = blk * items_per_blk + q + rb

                        @pl.when(j >= nb)
                        def _():
                            store_copy(j - nb, rb).wait()

                        gather_copy(ib, q + rb, rb).start()

                        @pl.when(j >= 2)
                        def _():
                            gather_copy(ib, q + rb, (rb - 2) % nb).wait()
                            store_copy(j - 2, (rb - 2) % nb).start()

        for k in (2, 1):
            j = steps - k
            rbk = j % nb
            gather_copy(0, 0, rbk).wait()
            store_copy(j, rbk).start()
        for rb in range(nb):
            store_copy(steps - 1, rb).wait()

    return gather_kernel(t2, idx_flat)


def kernel(category, table, W, b):
    batch, hist = category.shape
    t2 = _transform_table(table, W, b)
    idx = category.reshape(-1).astype(jnp.int32)
    out = _sc_gather(t2, idx)
    return out.reshape(batch, hist, W.shape[0])


# speedup vs baseline: 1.0028x; 1.0028x over previous
import functools

import jax
import jax.numpy as jnp
from jax import lax
from jax.experimental import pallas as pl
from jax.experimental.pallas import tpu as pltpu
from jax.experimental.pallas import tpu_sc as plsc

_ROWS_BLOCK = 10000
_GATHER_WINDOW = 128


def _transform_body(t_ref, w_ref, b_ref, o_ref):
    x = t_ref[...]
    row_ids = lax.broadcasted_iota(jnp.int32, x.shape, 0)
    is_row0 = jnp.logical_and(pl.program_id(0) == 0, row_ids == 0)
    x = jnp.where(is_row0, 0.0, x)
    y = lax.dot_general(
        x,
        w_ref[...],
        (((1,), (1,)), ((), ())),
        preferred_element_type=jnp.float32,
        precision=lax.Precision.DEFAULT,
    )
    o_ref[...] = jnp.maximum(y + b_ref[...], 0.0)


def _transform_table(table, W, b):
    num_rows, embed_dim = table.shape
    out_dim = W.shape[0]
    grid = num_rows // _ROWS_BLOCK
    return pl.pallas_call(
        _transform_body,
        grid=(grid,),
        in_specs=[
            pl.BlockSpec((_ROWS_BLOCK, embed_dim), lambda i: (i, 0)),
            pl.BlockSpec((out_dim, embed_dim), lambda i: (0, 0)),
            pl.BlockSpec((1, out_dim), lambda i: (0, 0)),
        ],
        out_specs=pl.BlockSpec((_ROWS_BLOCK, out_dim), lambda i: (i, 0)),
        out_shape=jax.ShapeDtypeStruct((num_rows, out_dim), jnp.float32),
    )(table, W, b.reshape(1, out_dim))


def _sc_gather(t2, idx_flat):
    n = idx_flat.shape[0]
    out_dim = t2.shape[1]
    num_workers = 32
    per_worker = n // num_workers
    chunk = _GATHER_WINDOW
    steps = per_worker // chunk
    idx_blk = 1024
    items_per_blk = idx_blk // chunk
    num_blks = per_worker // idx_blk
    mesh = plsc.VectorSubcoreMesh(core_axis_name="c", subcore_axis_name="s")

    @functools.partial(
        pl.kernel,
        out_type=jax.ShapeDtypeStruct((n, out_dim), jnp.float32),
        mesh=mesh,
        scratch_types=(
            [pltpu.VMEM((idx_blk,), jnp.int32)] * 4
            + [pltpu.VMEM((chunk, out_dim), jnp.float32)] * 4
            + [pltpu.SemaphoreType.DMA] * 12
        ),
    )
    def gather_kernel(t2_hbm, i_hbm, o_hbm,
                      idx0, idx1, idx2, idx3, rows0, rows1, rows2, rows3,
                      isem0, isem1, isem2, isem3,
                      gsem0, gsem1, gsem2, gsem3,
                      ssem0, ssem1, ssem2, ssem3):
        idx_bufs = (idx0, idx1, idx2, idx3)
        rows_bufs = (rows0, rows1, rows2, rows3)
        isems = (isem0, isem1, isem2, isem3)
        gsems = (gsem0, gsem1, gsem2, gsem3)
        ssems = (ssem0, ssem1, ssem2, ssem3)
        nb = 4
        wid = lax.axis_index("s") * 2 + lax.axis_index("c")
        base = wid * per_worker

        def idx_copy(blk, ib):
            return pltpu.make_async_copy(
                i_hbm.at[pl.ds(base + blk * idx_blk, idx_blk)],
                idx_bufs[ib], isems[ib])

        def gather_copy(ib, item_in_blk, rb):
            return pltpu.make_async_copy(
                t2_hbm.at[idx_bufs[ib].at[pl.ds(item_in_blk * chunk, chunk)]],
                rows_bufs[rb], gsems[rb])

        def store_copy(j, rb):
            return pltpu.make_async_copy(
                rows_bufs[rb], o_hbm.at[pl.ds(base + j * chunk, chunk)],
                ssems[rb])

        idx_copy(0, 0).start()

        @pl.loop(0, num_blks, step=4)
        def _(B):
            for ib in range(4):
                blk = B + ib
                idx_copy(blk, ib).wait()

                @pl.when(blk + 1 < num_blks)
                def _():
                    idx_copy(blk + 1, (ib + 1) % 4).start()

                @pl.loop(0, items_per_blk, step=nb)
                def _(q):
                    for rb in range(nb):
                        j = blk * items_per_blk + q + rb

                        @pl.when(j >= nb)
                        def _():
                            store_copy(j - nb, rb).wait()

                        gather_copy(ib, q + rb, rb).start()

                        @pl.when(j >= 2)
                        def _():
                            gather_copy(ib, q + rb, (rb - 2) % nb).wait()
                            store_copy(j - 2, (rb - 2) % nb).start()

        for k in (2, 1):
            j = steps - k
            rbk = j % nb
            gather_copy(0, 0, rbk).wait()
            store_copy(j, rbk).start()
        for rb in range(nb):
            store_copy(steps - 1, rb).wait()

    return gather_kernel(t2, idx_flat)


def kernel(category, table, W, b):
    batch, hist = category.shape
    t2 = _transform_table(table, W, b)
    idx = category.reshape(-1).astype(jnp.int32)
    out = _sc_gather(t2, idx)
    return out.reshape(batch, hist, W.shape[0])
